# baseline (speedup 1.0000x reference)
import jax
import jax.numpy as jnp
from jax import lax
from jax.experimental import pallas as pl
from jax.experimental.pallas import tpu as pltpu
from jax.experimental.pallas import tpu_sc as plsc

NN = 10000
NP = 10240
NE = 320000
D = 128
WSZ = 128
NWT = 159
NQ = NWT // 3
NEP = 16 * NWT * WSZ
HD = 64
ACC_W = 80


def _mm_body(prev, W, asrc, adst, h0, h1, ss, sd):
    hb = jnp.dot(prev[...], W[...], preferred_element_type=jnp.float32)
    h0[...] = hb[:, :HD]
    h1[...] = hb[:, HD:]
    ss[...] = jnp.sum(hb * asrc[...], axis=1)
    sd[...] = jnp.sum(hb * adst[...], axis=1)


_mm_call = pl.pallas_call(
    _mm_body,
    grid=(NP // 1024,),
    in_specs=[
        pl.BlockSpec((1024, D), lambda i: (i, 0)),
        pl.BlockSpec((D, D), lambda i: (0, 0)),
        pl.BlockSpec((1, D), lambda i: (0, 0)),
        pl.BlockSpec((1, D), lambda i: (0, 0)),
    ],
    out_specs=[
        pl.BlockSpec((1024, HD), lambda i: (i, 0)),
        pl.BlockSpec((1024, HD), lambda i: (i, 0)),
        pl.BlockSpec((1024,), lambda i: (i,)),
        pl.BlockSpec((1024,), lambda i: (i,)),
    ],
    out_shape=[
        jax.ShapeDtypeStruct((NP, HD), jnp.float32),
        jax.ShapeDtypeStruct((NP, HD), jnp.float32),
        jax.ShapeDtypeStruct((NP,), jnp.float32),
        jax.ShapeDtypeStruct((NP,), jnp.float32),
    ],
)


def _norm_bn(o0, o1, b, g, bb, m, v):
    U = jnp.concatenate([o0[:NN, :HD], o1[:NN, :HD]], axis=1)
    Dn = o0[:NN, HD:HD + 1]
    xi = jnp.maximum(U / (Dn + 1e-16) + b[...], 0.0)
    return (xi - m[...]) / jnp.sqrt(v[...] + 1e-5) * g[...] + bb[...]


def _fuse_body(o0, o1, b, g, bb, m, v, prev, pW, pb, W2, a2s, a2d,
               prevnew, h0, h1, ss, sd):
    xi = _norm_bn(o0, o1, b, g, bb, m, v)
    res = jnp.dot(prev[...], pW[...], preferred_element_type=jnp.float32)
    pn = jnp.concatenate(
        [xi + res[:NN, :] + pb[...], jnp.zeros((NP - NN, D), jnp.float32)], axis=0)
    prevnew[...] = pn
    hb = jnp.dot(pn, W2[...], preferred_element_type=jnp.float32)
    h0[...] = hb[:, :HD]
    h1[...] = hb[:, HD:]
    ss[...] = jnp.sum(hb * a2s[...], axis=1)
    sd[...] = jnp.sum(hb * a2d[...], axis=1)


_fuse_call = pl.pallas_call(
    _fuse_body,
    out_shape=[
        jax.ShapeDtypeStruct((NP, D), jnp.float32),
        jax.ShapeDtypeStruct((NP, HD), jnp.float32),
        jax.ShapeDtypeStruct((NP, HD), jnp.float32),
        jax.ShapeDtypeStruct((NP,), jnp.float32),
        jax.ShapeDtypeStruct((NP,), jnp.float32),
    ],
)


def _fuse5_body(o0, o1, b, g, bb, m, v, prev, pW, pb,
                w1, b1, hg, hbb, hm, hv, w2, b2, out):
    xi = _norm_bn(o0, o1, b, g, bb, m, v)
    res = jnp.dot(prev[...], pW[...], preferred_element_type=jnp.float32)
    pn = xi + res[:NN, :] + pb[...]
    gmean = jnp.sum(pn, axis=0, keepdims=True) / NN
    h = jnp.maximum(jnp.dot(gmean, w1[...], preferred_element_type=jnp.float32)
                    + b1[...], 0.0)
    h = (h - hm[...]) / jnp.sqrt(hv[...] + 1e-5) * hg[...] + hbb[...]
    out[...] = jnp.dot(h, w2[...], preferred_element_type=jnp.float32) + b2[...]


_fuse5_call = pl.pallas_call(
    _fuse5_body,
    out_shape=jax.ShapeDtypeStruct((1, 1), jnp.float32),
)


def _edge_body(h0_hbm, h1_hbm, ssrc_hbm, sdst_hbm, src_hbm, dst_hbm, out0, out1,
               ssrc_v, sdst_v,
               src0, src1, src2, dst0, dst1, dst2,
               rows0, rows1, rows2, upd0, upd1, upd2, eew, acc,
               semi0, semi1, semi2, semg0, semg1, semg2,
               sems0, sems1, sems2):
    c = lax.axis_index("c")
    s = lax.axis_index("s")
    SRC = [src0, src1, src2]
    DST = [dst0, dst1, dst2]
    ROWS = [rows0, rows1, rows2]
    UPD = [upd0, upd1, upd2]
    SEMI = [semi0, semi1, semi2]
    SEMG = [semg0, semg1, semg2]
    SEMS = [sems0, sems1, sems2]

    pltpu.sync_copy(ssrc_hbm, ssrc_v)
    pltpu.sync_copy(sdst_hbm, sdst_v)

    zero = jnp.zeros((16,), jnp.float32)

    @plsc.parallel_loop(0, WSZ, unroll=4)
    def _(i):
        for j in range(ACC_W // 16):
            upd0[i, pl.ds(j * 16, 16)] = zero

    zb = s * (NP // 16)
    for k in range(5):
        pltpu.sync_copy(upd0, acc.at[pl.ds(zb + k * 128, 128), :])
    plsc.subcore_barrier()

    lane0 = lax.iota(jnp.int32, 16) == 0
    tb = s * NWT

    def fetch_idx(w, j):
        b = (tb + w) * WSZ
        pltpu.async_copy(src_hbm.at[pl.ds(b, WSZ)], SRC[j], SEMI[j])
        pltpu.async_copy(dst_hbm.at[pl.ds(b, WSZ)], DST[j], SEMI[j])

    def wait_idx(j):
        pltpu.make_async_copy(src_hbm.at[pl.ds(0, WSZ)], SRC[j], SEMI[j]).wait()
        pltpu.make_async_copy(dst_hbm.at[pl.ds(0, WSZ)], DST[j], SEMI[j]).wait()

    def issue_gather(j):
        @pl.when(c == 0)
        def _():
            pltpu.async_copy(h0_hbm.at[SRC[j]], ROWS[j], SEMG[j])

        @pl.when(c == 1)
        def _():
            pltpu.async_copy(h1_hbm.at[SRC[j]], ROWS[j], SEMG[j])

    def wait_gather(j):
        pltpu.make_async_copy(h0_hbm.at[SRC[j]], ROWS[j], SEMG[j]).wait()

    def wait_scatter(j):
        pltpu.make_async_copy(UPD[j], acc.at[DST[j]], SEMS[j]).wait()

    def compute(j):
        srcX, dstX, rowsX, updX = SRC[j], DST[j], ROWS[j], UPD[j]
        for g in range(WSZ // 16):
            si = srcX[pl.ds(g * 16, 16)]
            di = dstX[pl.ds(g * 16, 16)]
            e = plsc.load_gather(ssrc_v, [si]) + plsc.load_gather(sdst_v, [di])
            e = jnp.maximum(e, 0.2 * e)
            eew[pl.ds(g * 16, 16)] = jnp.exp(e)

        for g in range(WSZ // 16):
            eeg = eew[pl.ds(g * 16, 16)]

            @plsc.parallel_loop(0, 16, unroll=4)
            def _(l):
                el = g * 16 + l
                eb = jnp.take(eeg, jnp.full((16,), l, jnp.int32))
                for j2 in range(HD // 16):
                    updX[el, pl.ds(j2 * 16, 16)] = eb * rowsX[el, pl.ds(j2 * 16, 16)]
                updX[el, pl.ds(HD, 16)] = jnp.where(lane0, eb, 0.0)

        pltpu.async_copy(updX, acc.at[dstX], SEMS[j], add=True)

    fetch_idx(0, 0)
    fetch_idx(1, 1)
    wait_idx(0)
    issue_gather(0)

    def triple(q, _):
        for k in range(3):
            jp = (k - 1) % 3
            jg = (k - 2) % 3

            if k == 0:
                @pl.when(q > 0)
                def _():
                    wait_scatter(jp)
            else:
                wait_scatter(jp)
            wf = 3 * q + k + 2
            if k == 0:
                fetch_idx(wf, jp)
            else:
                @pl.when(wf < NWT)
                def _():
                    fetch_idx(wf, jp)

            wg = 3 * q + k + 1
            if k == 2:
                @pl.when(wg < NWT)
                def _():
                    wait_idx(jg)
                    issue_gather(jg)
            else:
                wait_idx(jg)
                issue_gather(jg)

            wait_gather(k)
            compute(k)
        return 0

    lax.fori_loop(0, NQ, triple, 0)
    wait_scatter(2)
    plsc.subcore_barrier()

    for k in range(5):
        sl = pl.ds(zb + k * 128, 128)

        @pl.when(c == 0)
        def _():
            pltpu.sync_copy(acc.at[sl, :], out0.at[sl, :])

        @pl.when(c == 1)
        def _():
            pltpu.sync_copy(acc.at[sl, :], out1.at[sl, :])


_edge_call = pl.kernel(
    _edge_body,
    out_type=(
        jax.ShapeDtypeStruct((NP, ACC_W), jnp.float32),
        jax.ShapeDtypeStruct((NP, ACC_W), jnp.float32),
    ),
    mesh=plsc.VectorSubcoreMesh(core_axis_name="c", subcore_axis_name="s",
                                num_cores=2, num_subcores=16),
    compiler_params=pltpu.CompilerParams(needs_layout_passes=False,
                                         use_tc_tiling_on_sc=False),
    scratch_types=(
        [pltpu.VMEM((NP,), jnp.float32)] * 2
        + [pltpu.VMEM((WSZ,), jnp.int32)] * 6
        + [pltpu.VMEM((WSZ, HD), jnp.float32)] * 3
        + [pltpu.VMEM((WSZ, ACC_W), jnp.float32)] * 3
        + [pltpu.VMEM((WSZ,), jnp.float32)]
        + [pltpu.VMEM_SHARED((NP, ACC_W), jnp.float32)]
        + [pltpu.SemaphoreType.DMA] * 9
    ),
)


def kernel(x, edge_index, params):
    p = params
    pade = NEP - NE
    pidx = jnp.arange(pade, dtype=jnp.int32)
    src = jnp.concatenate([edge_index[0], (pidx * 97) % NN])
    dst = jnp.concatenate([edge_index[1], NN + (pidx % (NP - NN))])
    xp = jnp.zeros((NP, D), jnp.float32).at[:NN].set(x)

    r2 = lambda a: a.reshape(1, D)
    zW = jnp.zeros((D, D), jnp.float32)
    zb = jnp.zeros((1, D), jnp.float32)

    h0, h1, ss, sd = _mm_call(xp, p['conv1_W'], r2(p['conv1_asrc']), r2(p['conv1_adst']))
    prev = xp
    for i in range(1, 5):
        o0, o1 = _edge_call(h0, h1, ss, sd, src, dst)
        bn = (r2(p['bn%d_g' % i]), r2(p['bn%d_b' % i]),
              r2(p['bn%d_m' % i]), r2(p['bn%d_v' % i]))
        pW = zW if i == 1 else p['proj%d_W' % i]
        pb = zb if i == 1 else r2(p['proj%d_b' % i])
        j = i + 1
        prev, h0, h1, ss, sd = _fuse_call(
            o0, o1, r2(p['conv%d_b' % i]), *bn, prev, pW, pb,
            p['conv%d_W' % j], r2(p['conv%d_asrc' % j]), r2(p['conv%d_adst' % j]))

    o0, o1 = _edge_call(h0, h1, ss, sd, src, dst)
    bn5 = (r2(p['bn5_g']), r2(p['bn5_b']), r2(p['bn5_m']), r2(p['bn5_v']))
    out = _fuse5_call(o0, o1, r2(p['conv5_b']), *bn5, prev,
                      p['proj5_W'], r2(p['proj5_b']),
                      p['head_W1'], p['head_b1'][None, :],
                      p['headbn_g'][None, :], p['headbn_b'][None, :],
                      p['headbn_m'][None, :], p['headbn_v'][None, :],
                      p['head_W2'], p['head_b2'][None, :])
    return out.reshape(-1)


# speedup vs baseline: 1.3937x; 1.3937x over previous
import jax
import jax.numpy as jnp
from jax import lax
from jax.experimental import pallas as pl
from jax.experimental.pallas import tpu as pltpu
from jax.experimental.pallas import tpu_sc as plsc

NN = 10000
NP = 10240
NE = 320000
D = 128
WSZ = 128
NWT = 159
NQ = NWT // 3
NEP = 16 * NWT * WSZ
HD = 64
ACC_W = 80


def _mm_body(prev, W, asrc, adst, h0, h1, ss, sd):
    hb = jnp.dot(prev[...], W[...], preferred_element_type=jnp.float32)
    h0[...] = hb[:, :HD]
    h1[...] = hb[:, HD:]
    ss[...] = jnp.sum(hb * asrc[...], axis=1)
    sd[...] = jnp.sum(hb * adst[...], axis=1)


_mm_call = pl.pallas_call(
    _mm_body,
    grid=(NP // 1024,),
    in_specs=[
        pl.BlockSpec((1024, D), lambda i: (i, 0)),
        pl.BlockSpec((D, D), lambda i: (0, 0)),
        pl.BlockSpec((1, D), lambda i: (0, 0)),
        pl.BlockSpec((1, D), lambda i: (0, 0)),
    ],
    out_specs=[
        pl.BlockSpec((1024, HD), lambda i: (i, 0)),
        pl.BlockSpec((1024, HD), lambda i: (i, 0)),
        pl.BlockSpec((1024,), lambda i: (i,)),
        pl.BlockSpec((1024,), lambda i: (i,)),
    ],
    out_shape=[
        jax.ShapeDtypeStruct((NP, HD), jnp.float32),
        jax.ShapeDtypeStruct((NP, HD), jnp.float32),
        jax.ShapeDtypeStruct((NP,), jnp.float32),
        jax.ShapeDtypeStruct((NP,), jnp.float32),
    ],
)


def _norm_bn(o0, o1, b, g, bb, m, v, mask):
    U = jnp.concatenate([o0[:, :HD], o1[:, :HD]], axis=1)
    Dn = o0[:, HD:HD + 1]
    xi = jnp.maximum(U / (Dn + 1e-16) + b[...], 0.0)
    xi = (xi - m[...]) / jnp.sqrt(v[...] + 1e-5) * g[...] + bb[...]
    return jnp.where(mask, xi, 0.0)


def _row_mask(i):
    row = lax.broadcasted_iota(jnp.int32, (1024, 1), 0) + i * 1024
    return row < NN


def _fuse_body(o0, o1, b, g, bb, m, v, prev, pW, pb, W2, a2s, a2d,
               prevnew, h0, h1, ss, sd):
    i = pl.program_id(0)
    mask = _row_mask(i)
    xi = _norm_bn(o0[...], o1[...], b, g, bb, m, v, mask)
    res = jnp.dot(prev[...], pW[...], preferred_element_type=jnp.float32)
    pn = jnp.where(mask, xi + res + pb[...], 0.0)
    prevnew[...] = pn
    hb = jnp.dot(pn, W2[...], preferred_element_type=jnp.float32)
    h0[...] = hb[:, :HD]
    h1[...] = hb[:, HD:]
    ss[...] = jnp.sum(hb * a2s[...], axis=1)
    sd[...] = jnp.sum(hb * a2d[...], axis=1)


_bs_acc = pl.BlockSpec((1024, ACC_W), lambda i: (i, 0))
_bs_d = pl.BlockSpec((1024, D), lambda i: (i, 0))
_bs_v = pl.BlockSpec((1, D), lambda i: (0, 0))
_bs_w = pl.BlockSpec((D, D), lambda i: (0, 0))

_fuse_call = pl.pallas_call(
    _fuse_body,
    grid=(NP // 1024,),
    in_specs=[_bs_acc, _bs_acc] + [_bs_v] * 5 + [_bs_d, _bs_w, _bs_v, _bs_w,
                                                 _bs_v, _bs_v],
    out_specs=[
        _bs_d,
        pl.BlockSpec((1024, HD), lambda i: (i, 0)),
        pl.BlockSpec((1024, HD), lambda i: (i, 0)),
        pl.BlockSpec((1024,), lambda i: (i,)),
        pl.BlockSpec((1024,), lambda i: (i,)),
    ],
    out_shape=[
        jax.ShapeDtypeStruct((NP, D), jnp.float32),
        jax.ShapeDtypeStruct((NP, HD), jnp.float32),
        jax.ShapeDtypeStruct((NP, HD), jnp.float32),
        jax.ShapeDtypeStruct((NP,), jnp.float32),
        jax.ShapeDtypeStruct((NP,), jnp.float32),
    ],
)


def _fuse5_body(o0, o1, b, g, bb, m, v, prev, pW, pb,
                w1, b1, hg, hbb, hm, hv, w2, b2, out, gsum):
    i = pl.program_id(0)
    mask = _row_mask(i)
    xi = _norm_bn(o0[...], o1[...], b, g, bb, m, v, mask)
    res = jnp.dot(prev[...], pW[...], preferred_element_type=jnp.float32)
    pn = jnp.where(mask, xi + res + pb[...], 0.0)
    part = jnp.sum(pn, axis=0, keepdims=True)

    @pl.when(i == 0)
    def _():
        gsum[...] = jnp.zeros((1, D), jnp.float32)

    gsum[...] += part

    @pl.when(i == NP // 1024 - 1)
    def _():
        gmean = gsum[...] / NN
        h = jnp.maximum(jnp.dot(gmean, w1[...], preferred_element_type=jnp.float32)
                        + b1[...], 0.0)
        h = (h - hm[...]) / jnp.sqrt(hv[...] + 1e-5) * hg[...] + hbb[...]
        out[...] = jnp.dot(h, w2[...], preferred_element_type=jnp.float32) + b2[...]


_bs_h = pl.BlockSpec((D, HD), lambda i: (0, 0))
_fuse5_call = pl.pallas_call(
    _fuse5_body,
    grid=(NP // 1024,),
    in_specs=[_bs_acc, _bs_acc] + [_bs_v] * 5 + [_bs_d, _bs_w, _bs_v]
             + [_bs_h, pl.BlockSpec((1, HD), lambda i: (0, 0))]
             + [pl.BlockSpec((1, HD), lambda i: (0, 0))] * 4
             + [pl.BlockSpec((HD, 1), lambda i: (0, 0)),
                pl.BlockSpec((1, 1), lambda i: (0, 0))],
    out_specs=pl.BlockSpec((1, 1), lambda i: (0, 0)),
    out_shape=jax.ShapeDtypeStruct((1, 1), jnp.float32),
    scratch_shapes=[pltpu.VMEM((1, D), jnp.float32)],
)


def _edge_body(h0_hbm, h1_hbm, ssrc_hbm, sdst_hbm, src_hbm, dst_hbm, out0, out1,
               ssrc_v, sdst_v,
               src0, src1, src2, dst0, dst1, dst2,
               rows0, rows1, rows2, upd0, upd1, upd2, eew, acc,
               semi0, semi1, semi2, semg0, semg1, semg2,
               sems0, sems1, sems2):
    c = lax.axis_index("c")
    s = lax.axis_index("s")
    SRC = [src0, src1, src2]
    DST = [dst0, dst1, dst2]
    ROWS = [rows0, rows1, rows2]
    UPD = [upd0, upd1, upd2]
    SEMI = [semi0, semi1, semi2]
    SEMG = [semg0, semg1, semg2]
    SEMS = [sems0, sems1, sems2]

    pltpu.sync_copy(ssrc_hbm, ssrc_v)
    pltpu.sync_copy(sdst_hbm, sdst_v)

    zero = jnp.zeros((16,), jnp.float32)

    @plsc.parallel_loop(0, WSZ, unroll=4)
    def _(i):
        for j in range(ACC_W // 16):
            upd0[i, pl.ds(j * 16, 16)] = zero

    zb = s * (NP // 16)
    for k in range(5):
        pltpu.sync_copy(upd0, acc.at[pl.ds(zb + k * 128, 128), :])
    plsc.subcore_barrier()

    lane0 = lax.iota(jnp.int32, 16) == 0
    tb = s * NWT

    def fetch_idx(w, j):
        b = (tb + w) * WSZ
        pltpu.async_copy(src_hbm.at[pl.ds(b, WSZ)], SRC[j], SEMI[j])
        pltpu.async_copy(dst_hbm.at[pl.ds(b, WSZ)], DST[j], SEMI[j])

    def wait_idx(j):
        pltpu.make_async_copy(src_hbm.at[pl.ds(0, WSZ)], SRC[j], SEMI[j]).wait()
        pltpu.make_async_copy(dst_hbm.at[pl.ds(0, WSZ)], DST[j], SEMI[j]).wait()

    def issue_gather(j):
        @pl.when(c == 0)
        def _():
            pltpu.async_copy(h0_hbm.at[SRC[j]], ROWS[j], SEMG[j])

        @pl.when(c == 1)
        def _():
            pltpu.async_copy(h1_hbm.at[SRC[j]], ROWS[j], SEMG[j])

    def wait_gather(j):
        pltpu.make_async_copy(h0_hbm.at[SRC[j]], ROWS[j], SEMG[j]).wait()

    def wait_scatter(j):
        pltpu.make_async_copy(UPD[j], acc.at[DST[j]], SEMS[j]).wait()

    def compute(j):
        srcX, dstX, rowsX, updX = SRC[j], DST[j], ROWS[j], UPD[j]
        for g in range(WSZ // 16):
            si = srcX[pl.ds(g * 16, 16)]
            di = dstX[pl.ds(g * 16, 16)]
            e = plsc.load_gather(ssrc_v, [si]) + plsc.load_gather(sdst_v, [di])
            e = jnp.maximum(e, 0.2 * e)
            eew[pl.ds(g * 16, 16)] = jnp.exp(e)

        @plsc.parallel_loop(0, WSZ, unroll=4)
        def _(el):
            eb = plsc.load_gather(eew, [jnp.full((16,), el, jnp.int32)])
            for j2 in range(HD // 16):
                updX[el, pl.ds(j2 * 16, 16)] = eb * rowsX[el, pl.ds(j2 * 16, 16)]
            updX[el, pl.ds(HD, 16)] = jnp.where(lane0, eb, 0.0)

        pltpu.async_copy(updX, acc.at[dstX], SEMS[j], add=True)

    fetch_idx(0, 0)
    fetch_idx(1, 1)
    wait_idx(0)
    issue_gather(0)

    def triple(q, _):
        for k in range(3):
            jp = (k - 1) % 3
            jg = (k - 2) % 3

            if k == 0:
                @pl.when(q > 0)
                def _():
                    wait_scatter(jp)
            else:
                wait_scatter(jp)
            wf = 3 * q + k + 2
            if k == 0:
                fetch_idx(wf, jp)
            else:
                @pl.when(wf < NWT)
                def _():
                    fetch_idx(wf, jp)

            wg = 3 * q + k + 1
            if k == 2:
                @pl.when(wg < NWT)
                def _():
                    wait_idx(jg)
                    issue_gather(jg)
            else:
                wait_idx(jg)
                issue_gather(jg)

            wait_gather(k)
            compute(k)
        return 0

    lax.fori_loop(0, NQ, triple, 0)
    wait_scatter(2)
    plsc.subcore_barrier()

    for k in range(5):
        sl = pl.ds(zb + k * 128, 128)

        @pl.when(c == 0)
        def _():
            pltpu.sync_copy(acc.at[sl, :], out0.at[sl, :])

        @pl.when(c == 1)
        def _():
            pltpu.sync_copy(acc.at[sl, :], out1.at[sl, :])


_edge_call = pl.kernel(
    _edge_body,
    out_type=(
        jax.ShapeDtypeStruct((NP, ACC_W), jnp.float32),
        jax.ShapeDtypeStruct((NP, ACC_W), jnp.float32),
    ),
    mesh=plsc.VectorSubcoreMesh(core_axis_name="c", subcore_axis_name="s",
                                num_cores=2, num_subcores=16),
    compiler_params=pltpu.CompilerParams(needs_layout_passes=False,
                                         use_tc_tiling_on_sc=False),
    scratch_types=(
        [pltpu.VMEM((NP,), jnp.float32)] * 2
        + [pltpu.VMEM((WSZ,), jnp.int32)] * 6
        + [pltpu.VMEM((WSZ, HD), jnp.float32)] * 3
        + [pltpu.VMEM((WSZ, ACC_W), jnp.float32)] * 3
        + [pltpu.VMEM((WSZ,), jnp.float32)]
        + [pltpu.VMEM_SHARED((NP, ACC_W), jnp.float32)]
        + [pltpu.SemaphoreType.DMA] * 9
    ),
)


def kernel(x, edge_index, params):
    p = params
    pade = NEP - NE
    pidx = jnp.arange(pade, dtype=jnp.int32)
    src = jnp.concatenate([edge_index[0], (pidx * 97) % NN])
    dst = jnp.concatenate([edge_index[1], NN + (pidx % (NP - NN))])
    xp = jnp.zeros((NP, D), jnp.float32).at[:NN].set(x)

    r2 = lambda a: a.reshape(1, D)
    zW = jnp.zeros((D, D), jnp.float32)
    zb = jnp.zeros((1, D), jnp.float32)

    h0, h1, ss, sd = _mm_call(xp, p['conv1_W'], r2(p['conv1_asrc']), r2(p['conv1_adst']))
    prev = xp
    for i in range(1, 5):
        o0, o1 = _edge_call(h0, h1, ss, sd, src, dst)
        bn = (r2(p['bn%d_g' % i]), r2(p['bn%d_b' % i]),
              r2(p['bn%d_m' % i]), r2(p['bn%d_v' % i]))
        pW = zW if i == 1 else p['proj%d_W' % i]
        pb = zb if i == 1 else r2(p['proj%d_b' % i])
        j = i + 1
        prev, h0, h1, ss, sd = _fuse_call(
            o0, o1, r2(p['conv%d_b' % i]), *bn, prev, pW, pb,
            p['conv%d_W' % j], r2(p['conv%d_asrc' % j]), r2(p['conv%d_adst' % j]))

    o0, o1 = _edge_call(h0, h1, ss, sd, src, dst)
    bn5 = (r2(p['bn5_g']), r2(p['bn5_b']), r2(p['bn5_m']), r2(p['bn5_v']))
    out = _fuse5_call(o0, o1, r2(p['conv5_b']), *bn5, prev,
                      p['proj5_W'], r2(p['proj5_b']),
                      p['head_W1'], p['head_b1'][None, :],
                      p['headbn_g'][None, :], p['headbn_b'][None, :],
                      p['headbn_m'][None, :], p['headbn_v'][None, :],
                      p['head_W2'], p['head_b2'][None, :])
    return out.reshape(-1)


# speedup vs baseline: 1.3954x; 1.0012x over previous
import jax
import jax.numpy as jnp
from jax import lax
from jax.experimental import pallas as pl
from jax.experimental.pallas import tpu as pltpu
from jax.experimental.pallas import tpu_sc as plsc

NN = 10000
NP = 10240
NE = 320000
D = 128
WSZ = 128
NWT = 159
NQ = NWT // 3
NEP = 16 * NWT * WSZ
HD = 64
ACC_W = 80


def _mm_body(prev, W, asrc, adst, h0, h1, ss, sd):
    hb = jnp.dot(prev[...], W[...], preferred_element_type=jnp.float32)
    h0[...] = hb[:, :HD]
    h1[...] = hb[:, HD:]
    ss[...] = jnp.sum(hb * asrc[...], axis=1)
    sd[...] = jnp.sum(hb * adst[...], axis=1)


_mm_call = pl.pallas_call(
    _mm_body,
    grid=(NP // 1024,),
    in_specs=[
        pl.BlockSpec((1024, D), lambda i: (i, 0)),
        pl.BlockSpec((D, D), lambda i: (0, 0)),
        pl.BlockSpec((1, D), lambda i: (0, 0)),
        pl.BlockSpec((1, D), lambda i: (0, 0)),
    ],
    out_specs=[
        pl.BlockSpec((1024, HD), lambda i: (i, 0)),
        pl.BlockSpec((1024, HD), lambda i: (i, 0)),
        pl.BlockSpec((1024,), lambda i: (i,)),
        pl.BlockSpec((1024,), lambda i: (i,)),
    ],
    out_shape=[
        jax.ShapeDtypeStruct((NP, HD), jnp.float32),
        jax.ShapeDtypeStruct((NP, HD), jnp.float32),
        jax.ShapeDtypeStruct((NP,), jnp.float32),
        jax.ShapeDtypeStruct((NP,), jnp.float32),
    ],
)


def _norm_bn(o0, o1, b, g, bb, m, v, mask):
    U = jnp.concatenate([o0[:, :HD], o1[:, :HD]], axis=1)
    Dn = o0[:, HD:HD + 1]
    xi = jnp.maximum(U / (Dn + 1e-16) + b[...], 0.0)
    xi = (xi - m[...]) / jnp.sqrt(v[...] + 1e-5) * g[...] + bb[...]
    return jnp.where(mask, xi, 0.0)


def _row_mask(i):
    row = lax.broadcasted_iota(jnp.int32, (1024, 1), 0) + i * 1024
    return row < NN


def _fuse_body(o0, o1, b, g, bb, m, v, prev, pW, pb, W2, a2s, a2d,
               prevnew, h0, h1, ss, sd):
    i = pl.program_id(0)
    mask = _row_mask(i)
    xi = _norm_bn(o0[...], o1[...], b, g, bb, m, v, mask)
    res = jnp.dot(prev[...], pW[...], preferred_element_type=jnp.float32)
    pn = jnp.where(mask, xi + res + pb[...], 0.0)
    prevnew[...] = pn
    hb = jnp.dot(pn, W2[...], preferred_element_type=jnp.float32)
    h0[...] = hb[:, :HD]
    h1[...] = hb[:, HD:]
    ss[...] = jnp.sum(hb * a2s[...], axis=1)
    sd[...] = jnp.sum(hb * a2d[...], axis=1)


_bs_acc = pl.BlockSpec((1024, ACC_W), lambda i: (i, 0))
_bs_d = pl.BlockSpec((1024, D), lambda i: (i, 0))
_bs_v = pl.BlockSpec((1, D), lambda i: (0, 0))
_bs_w = pl.BlockSpec((D, D), lambda i: (0, 0))

_fuse_call = pl.pallas_call(
    _fuse_body,
    grid=(NP // 1024,),
    in_specs=[_bs_acc, _bs_acc] + [_bs_v] * 5 + [_bs_d, _bs_w, _bs_v, _bs_w,
                                                 _bs_v, _bs_v],
    out_specs=[
        _bs_d,
        pl.BlockSpec((1024, HD), lambda i: (i, 0)),
        pl.BlockSpec((1024, HD), lambda i: (i, 0)),
        pl.BlockSpec((1024,), lambda i: (i,)),
        pl.BlockSpec((1024,), lambda i: (i,)),
    ],
    out_shape=[
        jax.ShapeDtypeStruct((NP, D), jnp.float32),
        jax.ShapeDtypeStruct((NP, HD), jnp.float32),
        jax.ShapeDtypeStruct((NP, HD), jnp.float32),
        jax.ShapeDtypeStruct((NP,), jnp.float32),
        jax.ShapeDtypeStruct((NP,), jnp.float32),
    ],
)


def _fuse5_body(o0, o1, b, g, bb, m, v, prev, pW, pb,
                w1, b1, hg, hbb, hm, hv, w2, b2, out, gsum):
    i = pl.program_id(0)
    mask = _row_mask(i)
    xi = _norm_bn(o0[...], o1[...], b, g, bb, m, v, mask)
    res = jnp.dot(prev[...], pW[...], preferred_element_type=jnp.float32)
    pn = jnp.where(mask, xi + res + pb[...], 0.0)
    part = jnp.sum(pn, axis=0, keepdims=True)

    @pl.when(i == 0)
    def _():
        gsum[...] = jnp.zeros((1, D), jnp.float32)

    gsum[...] += part

    @pl.when(i == NP // 1024 - 1)
    def _():
        gmean = gsum[...] / NN
        h = jnp.maximum(jnp.dot(gmean, w1[...], preferred_element_type=jnp.float32)
                        + b1[...], 0.0)
        h = (h - hm[...]) / jnp.sqrt(hv[...] + 1e-5) * hg[...] + hbb[...]
        out[...] = jnp.dot(h, w2[...], preferred_element_type=jnp.float32) + b2[...]


_bs_h = pl.BlockSpec((D, HD), lambda i: (0, 0))
_fuse5_call = pl.pallas_call(
    _fuse5_body,
    grid=(NP // 1024,),
    in_specs=[_bs_acc, _bs_acc] + [_bs_v] * 5 + [_bs_d, _bs_w, _bs_v]
             + [_bs_h, pl.BlockSpec((1, HD), lambda i: (0, 0))]
             + [pl.BlockSpec((1, HD), lambda i: (0, 0))] * 4
             + [pl.BlockSpec((HD, 1), lambda i: (0, 0)),
                pl.BlockSpec((1, 1), lambda i: (0, 0))],
    out_specs=pl.BlockSpec((1, 1), lambda i: (0, 0)),
    out_shape=jax.ShapeDtypeStruct((1, 1), jnp.float32),
    scratch_shapes=[pltpu.VMEM((1, D), jnp.float32)],
)


def _edge_body(h0_hbm, h1_hbm, ssrc_hbm, sdst_hbm, src_hbm, dst_hbm, out0, out1,
               ssrc_v, sdst_v,
               src0, src1, src2, dst0, dst1, dst2,
               rows0, rows1, rows2, upd0, upd1, upd2, eew, acc,
               semi0, semi1, semi2, semg0, semg1, semg2,
               sems0, sems1, sems2):
    c = lax.axis_index("c")
    s = lax.axis_index("s")
    SRC = [src0, src1, src2]
    DST = [dst0, dst1, dst2]
    ROWS = [rows0, rows1, rows2]
    UPD = [upd0, upd1, upd2]
    SEMI = [semi0, semi1, semi2]
    SEMG = [semg0, semg1, semg2]
    SEMS = [sems0, sems1, sems2]

    pltpu.sync_copy(ssrc_hbm, ssrc_v)
    pltpu.sync_copy(sdst_hbm, sdst_v)

    zero = jnp.zeros((16,), jnp.float32)

    @plsc.parallel_loop(0, WSZ, unroll=4)
    def _(i):
        for j in range(ACC_W // 16):
            upd0[i, pl.ds(j * 16, 16)] = zero

    zb = s * (NP // 16)
    for k in range(5):
        pltpu.sync_copy(upd0, acc.at[pl.ds(zb + k * 128, 128), :])
    plsc.subcore_barrier()

    lane0 = lax.iota(jnp.int32, 16) == 0
    tb = s * NWT

    def fetch_idx(w, j):
        b = (tb + w) * WSZ
        pltpu.async_copy(src_hbm.at[pl.ds(b, WSZ)], SRC[j], SEMI[j])
        pltpu.async_copy(dst_hbm.at[pl.ds(b, WSZ)], DST[j], SEMI[j])

    def wait_idx(j):
        pltpu.make_async_copy(src_hbm.at[pl.ds(0, WSZ)], SRC[j], SEMI[j]).wait()
        pltpu.make_async_copy(dst_hbm.at[pl.ds(0, WSZ)], DST[j], SEMI[j]).wait()

    def issue_gather(j):
        @pl.when(c == 0)
        def _():
            pltpu.async_copy(h0_hbm.at[SRC[j]], ROWS[j], SEMG[j])

        @pl.when(c == 1)
        def _():
            pltpu.async_copy(h1_hbm.at[SRC[j]], ROWS[j], SEMG[j])

    def wait_gather(j):
        pltpu.make_async_copy(h0_hbm.at[SRC[j]], ROWS[j], SEMG[j]).wait()

    def wait_scatter(j):
        pltpu.make_async_copy(UPD[j], acc.at[DST[j]], SEMS[j]).wait()

    def compute(j):
        srcX, dstX, rowsX, updX = SRC[j], DST[j], ROWS[j], UPD[j]
        for g in range(WSZ // 16):
            si = srcX[pl.ds(g * 16, 16)]
            di = dstX[pl.ds(g * 16, 16)]
            e = plsc.load_gather(ssrc_v, [si]) + plsc.load_gather(sdst_v, [di])
            e = jnp.maximum(e, 0.2 * e)
            eew[pl.ds(g * 16, 16)] = jnp.exp(e)

        @plsc.parallel_loop(0, WSZ, unroll=8)
        def _(el):
            eb = plsc.load_gather(eew, [jnp.full((16,), el, jnp.int32)])
            for j2 in range(HD // 16):
                updX[el, pl.ds(j2 * 16, 16)] = eb * rowsX[el, pl.ds(j2 * 16, 16)]
            updX[el, pl.ds(HD, 16)] = jnp.where(lane0, eb, 0.0)

        pltpu.async_copy(updX, acc.at[dstX], SEMS[j], add=True)

    fetch_idx(0, 0)
    fetch_idx(1, 1)
    wait_idx(0)
    issue_gather(0)

    def triple(q, _):
        for k in range(3):
            jp = (k - 1) % 3
            jg = (k - 2) % 3

            if k == 0:
                @pl.when(q > 0)
                def _():
                    wait_scatter(jp)
            else:
                wait_scatter(jp)
            wf = 3 * q + k + 2
            if k == 0:
                fetch_idx(wf, jp)
            else:
                @pl.when(wf < NWT)
                def _():
                    fetch_idx(wf, jp)

            wg = 3 * q + k + 1
            if k == 2:
                @pl.when(wg < NWT)
                def _():
                    wait_idx(jg)
                    issue_gather(jg)
            else:
                wait_idx(jg)
                issue_gather(jg)

            wait_gather(k)
            compute(k)
        return 0

    lax.fori_loop(0, NQ, triple, 0)
    wait_scatter(2)
    plsc.subcore_barrier()

    for k in range(5):
        sl = pl.ds(zb + k * 128, 128)

        @pl.when(c == 0)
        def _():
            pltpu.sync_copy(acc.at[sl, :], out0.at[sl, :])

        @pl.when(c == 1)
        def _():
            pltpu.sync_copy(acc.at[sl, :], out1.at[sl, :])


_edge_call = pl.kernel(
    _edge_body,
    out_type=(
        jax.ShapeDtypeStruct((NP, ACC_W), jnp.float32),
        jax.ShapeDtypeStruct((NP, ACC_W), jnp.float32),
    ),
    mesh=plsc.VectorSubcoreMesh(core_axis_name="c", subcore_axis_name="s",
                                num_cores=2, num_subcores=16),
    compiler_params=pltpu.CompilerParams(needs_layout_passes=False,
                                         use_tc_tiling_on_sc=False),
    scratch_types=(
        [pltpu.VMEM((NP,), jnp.float32)] * 2
        + [pltpu.VMEM((WSZ,), jnp.int32)] * 6
        + [pltpu.VMEM((WSZ, HD), jnp.float32)] * 3
        + [pltpu.VMEM((WSZ, ACC_W), jnp.float32)] * 3
        + [pltpu.VMEM((WSZ,), jnp.float32)]
        + [pltpu.VMEM_SHARED((NP, ACC_W), jnp.float32)]
        + [pltpu.SemaphoreType.DMA] * 9
    ),
)


def kernel(x, edge_index, params):
    p = params
    pade = NEP - NE
    pidx = jnp.arange(pade, dtype=jnp.int32)
    src = jnp.concatenate([edge_index[0], (pidx * 97) % NN])
    dst = jnp.concatenate([edge_index[1], NN + (pidx % (NP - NN))])
    xp = jnp.zeros((NP, D), jnp.float32).at[:NN].set(x)

    r2 = lambda a: a.reshape(1, D)
    zW = jnp.zeros((D, D), jnp.float32)
    zb = jnp.zeros((1, D), jnp.float32)

    h0, h1, ss, sd = _mm_call(xp, p['conv1_W'], r2(p['conv1_asrc']), r2(p['conv1_adst']))
    prev = xp
    for i in range(1, 5):
        o0, o1 = _edge_call(h0, h1, ss, sd, src, dst)
        bn = (r2(p['bn%d_g' % i]), r2(p['bn%d_b' % i]),
              r2(p['bn%d_m' % i]), r2(p['bn%d_v' % i]))
        pW = zW if i == 1 else p['proj%d_W' % i]
        pb = zb if i == 1 else r2(p['proj%d_b' % i])
        j = i + 1
        prev, h0, h1, ss, sd = _fuse_call(
            o0, o1, r2(p['conv%d_b' % i]), *bn, prev, pW, pb,
            p['conv%d_W' % j], r2(p['conv%d_asrc' % j]), r2(p['conv%d_adst' % j]))

    o0, o1 = _edge_call(h0, h1, ss, sd, src, dst)
    bn5 = (r2(p['bn5_g']), r2(p['bn5_b']), r2(p['bn5_m']), r2(p['bn5_v']))
    out = _fuse5_call(o0, o1, r2(p['conv5_b']), *bn5, prev,
                      p['proj5_W'], r2(p['proj5_b']),
                      p['head_W1'], p['head_b1'][None, :],
                      p['headbn_g'][None, :], p['headbn_b'][None, :],
                      p['headbn_m'][None, :], p['headbn_v'][None, :],
                      p['head_W2'], p['head_b2'][None, :])
    return out.reshape(-1)


# speedup vs baseline: 1.4176x; 1.0159x over previous
import jax
import jax.numpy as jnp
from jax import lax
from jax.experimental import pallas as pl
from jax.experimental.pallas import tpu as pltpu
from jax.experimental.pallas import tpu_sc as plsc

NN = 10000
NP = 10240
NE = 320000
D = 128
WSZ = 128
NWT = 160
WPB = 8
NB = NWT // WPB
NEP = 16 * NWT * WSZ
HD = 64
ACC_W = 80


def _mm_body(prev, W, asrc, adst, h0, h1, ss, sd):
    hb = jnp.dot(prev[...], W[...], preferred_element_type=jnp.float32)
    h0[...] = hb[:, :HD]
    h1[...] = hb[:, HD:]
    ss[...] = jnp.sum(hb * asrc[...], axis=1)
    sd[...] = jnp.sum(hb * adst[...], axis=1)


_mm_call = pl.pallas_call(
    _mm_body,
    grid=(NP // 1024,),
    in_specs=[
        pl.BlockSpec((1024, D), lambda i: (i, 0)),
        pl.BlockSpec((D, D), lambda i: (0, 0)),
        pl.BlockSpec((1, D), lambda i: (0, 0)),
        pl.BlockSpec((1, D), lambda i: (0, 0)),
    ],
    out_specs=[
        pl.BlockSpec((1024, HD), lambda i: (i, 0)),
        pl.BlockSpec((1024, HD), lambda i: (i, 0)),
        pl.BlockSpec((1024,), lambda i: (i,)),
        pl.BlockSpec((1024,), lambda i: (i,)),
    ],
    out_shape=[
        jax.ShapeDtypeStruct((NP, HD), jnp.float32),
        jax.ShapeDtypeStruct((NP, HD), jnp.float32),
        jax.ShapeDtypeStruct((NP,), jnp.float32),
        jax.ShapeDtypeStruct((NP,), jnp.float32),
    ],
)


def _norm_bn(o0, o1, b, g, bb, m, v, mask):
    U = jnp.concatenate([o0[:, :HD], o1[:, :HD]], axis=1)
    Dn = o0[:, HD:HD + 1]
    xi = jnp.maximum(U / (Dn + 1e-16) + b[...], 0.0)
    xi = (xi - m[...]) / jnp.sqrt(v[...] + 1e-5) * g[...] + bb[...]
    return jnp.where(mask, xi, 0.0)


def _row_mask(i):
    row = lax.broadcasted_iota(jnp.int32, (1024, 1), 0) + i * 1024
    return row < NN


def _fuse_body(o0, o1, b, g, bb, m, v, prev, pW, pb, W2, a2s, a2d,
               prevnew, h0, h1, ss, sd):
    i = pl.program_id(0)
    mask = _row_mask(i)
    xi = _norm_bn(o0[...], o1[...], b, g, bb, m, v, mask)
    res = jnp.dot(prev[...], pW[...], preferred_element_type=jnp.float32)
    pn = jnp.where(mask, xi + res + pb[...], 0.0)
    prevnew[...] = pn
    hb = jnp.dot(pn, W2[...], preferred_element_type=jnp.float32)
    h0[...] = hb[:, :HD]
    h1[...] = hb[:, HD:]
    ss[...] = jnp.sum(hb * a2s[...], axis=1)
    sd[...] = jnp.sum(hb * a2d[...], axis=1)


_bs_acc = pl.BlockSpec((1024, ACC_W), lambda i: (i, 0))
_bs_d = pl.BlockSpec((1024, D), lambda i: (i, 0))
_bs_v = pl.BlockSpec((1, D), lambda i: (0, 0))
_bs_w = pl.BlockSpec((D, D), lambda i: (0, 0))

_fuse_call = pl.pallas_call(
    _fuse_body,
    grid=(NP // 1024,),
    in_specs=[_bs_acc, _bs_acc] + [_bs_v] * 5 + [_bs_d, _bs_w, _bs_v, _bs_w,
                                                 _bs_v, _bs_v],
    out_specs=[
        _bs_d,
        pl.BlockSpec((1024, HD), lambda i: (i, 0)),
        pl.BlockSpec((1024, HD), lambda i: (i, 0)),
        pl.BlockSpec((1024,), lambda i: (i,)),
        pl.BlockSpec((1024,), lambda i: (i,)),
    ],
    out_shape=[
        jax.ShapeDtypeStruct((NP, D), jnp.float32),
        jax.ShapeDtypeStruct((NP, HD), jnp.float32),
        jax.ShapeDtypeStruct((NP, HD), jnp.float32),
        jax.ShapeDtypeStruct((NP,), jnp.float32),
        jax.ShapeDtypeStruct((NP,), jnp.float32),
    ],
)


def _fuse5_body(o0, o1, b, g, bb, m, v, prev, pW, pb,
                w1, b1, hg, hbb, hm, hv, w2, b2, out, gsum):
    i = pl.program_id(0)
    mask = _row_mask(i)
    xi = _norm_bn(o0[...], o1[...], b, g, bb, m, v, mask)
    res = jnp.dot(prev[...], pW[...], preferred_element_type=jnp.float32)
    pn = jnp.where(mask, xi + res + pb[...], 0.0)
    part = jnp.sum(pn, axis=0, keepdims=True)

    @pl.when(i == 0)
    def _():
        gsum[...] = jnp.zeros((1, D), jnp.float32)

    gsum[...] += part

    @pl.when(i == NP // 1024 - 1)
    def _():
        gmean = gsum[...] / NN
        h = jnp.maximum(jnp.dot(gmean, w1[...], preferred_element_type=jnp.float32)
                        + b1[...], 0.0)
        h = (h - hm[...]) / jnp.sqrt(hv[...] + 1e-5) * hg[...] + hbb[...]
        out[...] = jnp.dot(h, w2[...], preferred_element_type=jnp.float32) + b2[...]


_bs_h = pl.BlockSpec((D, HD), lambda i: (0, 0))
_fuse5_call = pl.pallas_call(
    _fuse5_body,
    grid=(NP // 1024,),
    in_specs=[_bs_acc, _bs_acc] + [_bs_v] * 5 + [_bs_d, _bs_w, _bs_v]
             + [_bs_h, pl.BlockSpec((1, HD), lambda i: (0, 0))]
             + [pl.BlockSpec((1, HD), lambda i: (0, 0))] * 4
             + [pl.BlockSpec((HD, 1), lambda i: (0, 0)),
                pl.BlockSpec((1, 1), lambda i: (0, 0))],
    out_specs=pl.BlockSpec((1, 1), lambda i: (0, 0)),
    out_shape=jax.ShapeDtypeStruct((1, 1), jnp.float32),
    scratch_shapes=[pltpu.VMEM((1, D), jnp.float32)],
)


def _edge_body(h0_hbm, h1_hbm, ssrc_hbm, sdst_hbm, src_hbm, dst_hbm, out0, out1,
               ssrc_v, sdst_v, srcb, dstb, dstS0, dstS1, dstS2,
               rows0, rows1, rows2, upd0, upd1, upd2, eew, acc,
               semi, semg0, semg1, semg2, sems0, sems1, sems2):
    c = lax.axis_index("c")
    s = lax.axis_index("s")
    DSTS = [dstS0, dstS1, dstS2]
    ROWS = [rows0, rows1, rows2]
    UPD = [upd0, upd1, upd2]
    SEMG = [semg0, semg1, semg2]
    SEMS = [sems0, sems1, sems2]

    pltpu.sync_copy(ssrc_hbm, ssrc_v)
    pltpu.sync_copy(sdst_hbm, sdst_v)

    zero = jnp.zeros((16,), jnp.float32)

    @plsc.parallel_loop(0, WSZ, unroll=4)
    def _(i):
        for j in range(ACC_W // 16):
            upd0[i, pl.ds(j * 16, 16)] = zero

    zb = s * (NP // 16)
    for k in range(5):
        pltpu.sync_copy(upd0, acc.at[pl.ds(zb + k * 128, 128), :])
    plsc.subcore_barrier()

    lane0 = lax.iota(jnp.int32, 16) == 0
    tb = s * NWT * WSZ

    def issue_gather(w, j):
        sl = srcb.at[pl.ds(w * WSZ, WSZ)]

        @pl.when(c == 0)
        def _():
            pltpu.async_copy(h0_hbm.at[sl], ROWS[j], SEMG[j])

        @pl.when(c == 1)
        def _():
            pltpu.async_copy(h1_hbm.at[sl], ROWS[j], SEMG[j])

    def wait_gather(w, j):
        sl = srcb.at[pl.ds(w * WSZ, WSZ)]
        pltpu.make_async_copy(h0_hbm.at[sl], ROWS[j], SEMG[j]).wait()

    def wait_scatter(j):
        pltpu.make_async_copy(UPD[j], acc.at[DSTS[j]], SEMS[j]).wait()

    def block(blk, _):
        bb = tb + blk * (WPB * WSZ)
        d1 = pltpu.async_copy(src_hbm.at[pl.ds(bb, WPB * WSZ)], srcb, semi)
        d2 = pltpu.async_copy(dst_hbm.at[pl.ds(bb, WPB * WSZ)], dstb, semi)
        d1.wait()
        d2.wait()
        issue_gather(0, 0)
        issue_gather(1, 1)
        for w in range(WPB):
            j = w % 3
            dstX, rowsX, updX = DSTS[j], ROWS[j], UPD[j]
            for g in range(WSZ // 16):
                si = srcb[pl.ds(w * WSZ + g * 16, 16)]
                di = dstb[pl.ds(w * WSZ + g * 16, 16)]
                e = plsc.load_gather(ssrc_v, [si]) + plsc.load_gather(sdst_v, [di])
                e = jnp.maximum(e, 0.2 * e)
                eew[pl.ds(g * 16, 16)] = jnp.exp(e)
                dstX[pl.ds(g * 16, 16)] = di

            if w < 3:
                @pl.when(blk > 0)
                def _():
                    wait_scatter(j)
            else:
                wait_scatter(j)
            wait_gather(w, j)

            @plsc.parallel_loop(0, WSZ, unroll=8)
            def _(el):
                eb = plsc.load_gather(eew, [jnp.full((16,), el, jnp.int32)])
                for j2 in range(HD // 16):
                    updX[el, pl.ds(j2 * 16, 16)] = eb * rowsX[el, pl.ds(j2 * 16, 16)]
                updX[el, pl.ds(HD, 16)] = jnp.where(lane0, eb, 0.0)

            pltpu.async_copy(updX, acc.at[dstX], SEMS[j], add=True)
            if w + 2 < WPB:
                issue_gather(w + 2, (w + 2) % 3)
        return 0

    lax.fori_loop(0, NB, block, 0)
    wait_scatter(0)
    wait_scatter(1)
    wait_scatter(2)
    plsc.subcore_barrier()

    for k in range(5):
        sl = pl.ds(zb + k * 128, 128)

        @pl.when(c == 0)
        def _():
            pltpu.sync_copy(acc.at[sl, :], out0.at[sl, :])

        @pl.when(c == 1)
        def _():
            pltpu.sync_copy(acc.at[sl, :], out1.at[sl, :])


_edge_call = pl.kernel(
    _edge_body,
    out_type=(
        jax.ShapeDtypeStruct((NP, ACC_W), jnp.float32),
        jax.ShapeDtypeStruct((NP, ACC_W), jnp.float32),
    ),
    mesh=plsc.VectorSubcoreMesh(core_axis_name="c", subcore_axis_name="s",
                                num_cores=2, num_subcores=16),
    compiler_params=pltpu.CompilerParams(needs_layout_passes=False,
                                         use_tc_tiling_on_sc=False),
    scratch_types=(
        [pltpu.VMEM((NP,), jnp.float32)] * 2
        + [pltpu.VMEM((WPB * WSZ,), jnp.int32)] * 2
        + [pltpu.VMEM((WSZ,), jnp.int32)] * 3
        + [pltpu.VMEM((WSZ, HD), jnp.float32)] * 3
        + [pltpu.VMEM((WSZ, ACC_W), jnp.float32)] * 3
        + [pltpu.VMEM((WSZ,), jnp.float32)]
        + [pltpu.VMEM_SHARED((NP, ACC_W), jnp.float32)]
        + [pltpu.SemaphoreType.DMA] * 7
    ),
)


def kernel(x, edge_index, params):
    p = params
    pade = NEP - NE
    pidx = jnp.arange(pade, dtype=jnp.int32)
    src = jnp.concatenate([edge_index[0], (pidx * 97) % NN])
    dst = jnp.concatenate([edge_index[1], NN + (pidx % (NP - NN))])
    xp = jnp.zeros((NP, D), jnp.float32).at[:NN].set(x)

    r2 = lambda a: a.reshape(1, D)
    zW = jnp.zeros((D, D), jnp.float32)
    zb = jnp.zeros((1, D), jnp.float32)

    h0, h1, ss, sd = _mm_call(xp, p['conv1_W'], r2(p['conv1_asrc']), r2(p['conv1_adst']))
    prev = xp
    for i in range(1, 5):
        o0, o1 = _edge_call(h0, h1, ss, sd, src, dst)
        bn = (r2(p['bn%d_g' % i]), r2(p['bn%d_b' % i]),
              r2(p['bn%d_m' % i]), r2(p['bn%d_v' % i]))
        pW = zW if i == 1 else p['proj%d_W' % i]
        pb = zb if i == 1 else r2(p['proj%d_b' % i])
        j = i + 1
        prev, h0, h1, ss, sd = _fuse_call(
            o0, o1, r2(p['conv%d_b' % i]), *bn, prev, pW, pb,
            p['conv%d_W' % j], r2(p['conv%d_asrc' % j]), r2(p['conv%d_adst' % j]))

    o0, o1 = _edge_call(h0, h1, ss, sd, src, dst)
    bn5 = (r2(p['bn5_g']), r2(p['bn5_b']), r2(p['bn5_m']), r2(p['bn5_v']))
    out = _fuse5_call(o0, o1, r2(p['conv5_b']), *bn5, prev,
                      p['proj5_W'], r2(p['proj5_b']),
                      p['head_W1'], p['head_b1'][None, :],
                      p['headbn_g'][None, :], p['headbn_b'][None, :],
                      p['headbn_m'][None, :], p['headbn_v'][None, :],
                      p['head_W2'], p['head_b2'][None, :])
    return out.reshape(-1)


# speedup vs baseline: 1.4199x; 1.0016x over previous
import jax
import jax.numpy as jnp
from jax import lax
from jax.experimental import pallas as pl
from jax.experimental.pallas import tpu as pltpu
from jax.experimental.pallas import tpu_sc as plsc

NN = 10000
NP = 10240
NE = 320000
D = 128
WSZ = 128
NWT = 160
WPB = 8
NB = NWT // WPB
NEP = 16 * NWT * WSZ
HD = 64
ACC_W = 80


def _mm_body(prev, W, asrc, adst, h0, h1, ss, sd):
    hb = jnp.dot(prev[...], W[...], preferred_element_type=jnp.float32)
    h0[...] = hb[:, :HD]
    h1[...] = hb[:, HD:]
    ss[...] = jnp.sum(hb * asrc[...], axis=1)
    sd[...] = jnp.sum(hb * adst[...], axis=1)


_mm_call = pl.pallas_call(
    _mm_body,
    grid=(NP // 1024,),
    in_specs=[
        pl.BlockSpec((1024, D), lambda i: (i, 0)),
        pl.BlockSpec((D, D), lambda i: (0, 0)),
        pl.BlockSpec((1, D), lambda i: (0, 0)),
        pl.BlockSpec((1, D), lambda i: (0, 0)),
    ],
    out_specs=[
        pl.BlockSpec((1024, HD), lambda i: (i, 0)),
        pl.BlockSpec((1024, HD), lambda i: (i, 0)),
        pl.BlockSpec((1024,), lambda i: (i,)),
        pl.BlockSpec((1024,), lambda i: (i,)),
    ],
    out_shape=[
        jax.ShapeDtypeStruct((NP, HD), jnp.float32),
        jax.ShapeDtypeStruct((NP, HD), jnp.float32),
        jax.ShapeDtypeStruct((NP,), jnp.float32),
        jax.ShapeDtypeStruct((NP,), jnp.float32),
    ],
)


def _norm_bn(o0, o1, b, g, bb, m, v, mask):
    U = jnp.concatenate([o0[:, :HD], o1[:, :HD]], axis=1)
    Dn = o0[:, HD:HD + 1]
    xi = jnp.maximum(U / (Dn + 1e-16) + b[...], 0.0)
    xi = (xi - m[...]) / jnp.sqrt(v[...] + 1e-5) * g[...] + bb[...]
    return jnp.where(mask, xi, 0.0)


def _row_mask(i):
    row = lax.broadcasted_iota(jnp.int32, (1024, 1), 0) + i * 1024
    return row < NN


def _fuse_body(o0, o1, b, g, bb, m, v, prev, pW, pb, W2, a2s, a2d,
               prevnew, h0, h1, ss, sd):
    i = pl.program_id(0)
    mask = _row_mask(i)
    xi = _norm_bn(o0[...], o1[...], b, g, bb, m, v, mask)
    res = jnp.dot(prev[...], pW[...], preferred_element_type=jnp.float32)
    pn = jnp.where(mask, xi + res + pb[...], 0.0)
    prevnew[...] = pn
    hb = jnp.dot(pn, W2[...], preferred_element_type=jnp.float32)
    h0[...] = hb[:, :HD]
    h1[...] = hb[:, HD:]
    ss[...] = jnp.sum(hb * a2s[...], axis=1)
    sd[...] = jnp.sum(hb * a2d[...], axis=1)


_bs_acc = pl.BlockSpec((1024, ACC_W), lambda i: (i, 0))
_bs_d = pl.BlockSpec((1024, D), lambda i: (i, 0))
_bs_v = pl.BlockSpec((1, D), lambda i: (0, 0))
_bs_w = pl.BlockSpec((D, D), lambda i: (0, 0))

_fuse_call = pl.pallas_call(
    _fuse_body,
    grid=(NP // 1024,),
    in_specs=[_bs_acc, _bs_acc] + [_bs_v] * 5 + [_bs_d, _bs_w, _bs_v, _bs_w,
                                                 _bs_v, _bs_v],
    out_specs=[
        _bs_d,
        pl.BlockSpec((1024, HD), lambda i: (i, 0)),
        pl.BlockSpec((1024, HD), lambda i: (i, 0)),
        pl.BlockSpec((1024,), lambda i: (i,)),
        pl.BlockSpec((1024,), lambda i: (i,)),
    ],
    out_shape=[
        jax.ShapeDtypeStruct((NP, D), jnp.float32),
        jax.ShapeDtypeStruct((NP, HD), jnp.float32),
        jax.ShapeDtypeStruct((NP, HD), jnp.float32),
        jax.ShapeDtypeStruct((NP,), jnp.float32),
        jax.ShapeDtypeStruct((NP,), jnp.float32),
    ],
)


def _fuse5_body(o0, o1, b, g, bb, m, v, prev, pW, pb,
                w1, b1, hg, hbb, hm, hv, w2, b2, out, gsum):
    i = pl.program_id(0)
    mask = _row_mask(i)
    xi = _norm_bn(o0[...], o1[...], b, g, bb, m, v, mask)
    res = jnp.dot(prev[...], pW[...], preferred_element_type=jnp.float32)
    pn = jnp.where(mask, xi + res + pb[...], 0.0)
    part = jnp.sum(pn, axis=0, keepdims=True)

    @pl.when(i == 0)
    def _():
        gsum[...] = jnp.zeros((1, D), jnp.float32)

    gsum[...] += part

    @pl.when(i == NP // 1024 - 1)
    def _():
        gmean = gsum[...] / NN
        h = jnp.maximum(jnp.dot(gmean, w1[...], preferred_element_type=jnp.float32)
                        + b1[...], 0.0)
        h = (h - hm[...]) / jnp.sqrt(hv[...] + 1e-5) * hg[...] + hbb[...]
        out[...] = jnp.dot(h, w2[...], preferred_element_type=jnp.float32) + b2[...]


_bs_h = pl.BlockSpec((D, HD), lambda i: (0, 0))
_fuse5_call = pl.pallas_call(
    _fuse5_body,
    grid=(NP // 1024,),
    in_specs=[_bs_acc, _bs_acc] + [_bs_v] * 5 + [_bs_d, _bs_w, _bs_v]
             + [_bs_h, pl.BlockSpec((1, HD), lambda i: (0, 0))]
             + [pl.BlockSpec((1, HD), lambda i: (0, 0))] * 4
             + [pl.BlockSpec((HD, 1), lambda i: (0, 0)),
                pl.BlockSpec((1, 1), lambda i: (0, 0))],
    out_specs=pl.BlockSpec((1, 1), lambda i: (0, 0)),
    out_shape=jax.ShapeDtypeStruct((1, 1), jnp.float32),
    scratch_shapes=[pltpu.VMEM((1, D), jnp.float32)],
)


def _edge_body(h0_hbm, h1_hbm, ssrc_hbm, sdst_hbm, src_hbm, dst_hbm, out0, out1,
               ssrc_v, sdst_v, srcb, dstb, dstS0, dstS1, dstS2,
               rows0, rows1, rows2, upd0, upd1, upd2, eew, acc,
               semi, semg0, semg1, semg2, sems0, sems1, sems2):
    c = lax.axis_index("c")
    s = lax.axis_index("s")
    DSTS = [dstS0, dstS1, dstS2]
    ROWS = [rows0, rows1, rows2]
    UPD = [upd0, upd1, upd2]
    SEMG = [semg0, semg1, semg2]
    SEMS = [sems0, sems1, sems2]

    pltpu.sync_copy(ssrc_hbm, ssrc_v)
    pltpu.sync_copy(sdst_hbm, sdst_v)

    zero = jnp.zeros((16,), jnp.float32)

    @plsc.parallel_loop(0, WSZ, unroll=4)
    def _(i):
        for j in range(ACC_W // 16):
            upd0[i, pl.ds(j * 16, 16)] = zero

    zb = s * (NP // 16)
    for k in range(5):
        pltpu.sync_copy(upd0, acc.at[pl.ds(zb + k * 128, 128), :])
    plsc.subcore_barrier()

    lane0 = lax.iota(jnp.int32, 16) == 0
    tb = s * NWT * WSZ

    def issue_gather(w, j):
        sl = srcb.at[pl.ds(w * WSZ, WSZ)]

        @pl.when(c == 0)
        def _():
            pltpu.async_copy(h0_hbm.at[sl], ROWS[j], SEMG[j])

        @pl.when(c == 1)
        def _():
            pltpu.async_copy(h1_hbm.at[sl], ROWS[j], SEMG[j])

    def wait_gather(w, j):
        sl = srcb.at[pl.ds(w * WSZ, WSZ)]
        pltpu.make_async_copy(h0_hbm.at[sl], ROWS[j], SEMG[j]).wait()

    def wait_scatter(j):
        pltpu.make_async_copy(UPD[j], acc.at[DSTS[j]], SEMS[j]).wait()

    def block(blk, _):
        bb = tb + blk * (WPB * WSZ)
        d1 = pltpu.async_copy(src_hbm.at[pl.ds(bb, WPB * WSZ)], srcb, semi)
        d2 = pltpu.async_copy(dst_hbm.at[pl.ds(bb, WPB * WSZ)], dstb, semi)
        d1.wait()
        d2.wait()
        issue_gather(0, 0)
        issue_gather(1, 1)
        for w in range(WPB):
            j = w % 3
            dstX, rowsX, updX = DSTS[j], ROWS[j], UPD[j]
            for g in range(WSZ // 16):
                si = srcb[pl.ds(w * WSZ + g * 16, 16)]
                di = dstb[pl.ds(w * WSZ + g * 16, 16)]
                e = plsc.load_gather(ssrc_v, [si]) + plsc.load_gather(sdst_v, [di])
                e = jnp.maximum(e, 0.2 * e)
                eew[pl.ds(g * 16, 16)] = jnp.exp(e)
                dstX[pl.ds(g * 16, 16)] = di

            if w < 3:
                @pl.when(blk > 0)
                def _():
                    wait_scatter(j)
            else:
                wait_scatter(j)
            wait_gather(w, j)

            @plsc.parallel_loop(0, WSZ, unroll=8)
            def _(el):
                eb = plsc.load_gather(eew, [jnp.full((16,), el, jnp.int32)])
                for j2 in range(HD // 16):
                    updX[el, pl.ds(j2 * 16, 16)] = eb * rowsX[el, pl.ds(j2 * 16, 16)]
                updX[el, pl.ds(HD, 16)] = jnp.where(lane0, eb, 0.0)

            pltpu.async_copy(updX, acc.at[dstX], SEMS[j], add=True)
            if w + 2 < WPB:
                issue_gather(w + 2, (w + 2) % 3)
        return 0

    lax.fori_loop(0, NB, block, 0)
    wait_scatter(0)
    wait_scatter(1)
    wait_scatter(2)
    plsc.subcore_barrier()

    for k in range(5):
        sl = pl.ds(zb + k * 128, 128)

        @pl.when(c == 0)
        def _():
            pltpu.sync_copy(acc.at[sl, :], out0.at[sl, :])

        @pl.when(c == 1)
        def _():
            pltpu.sync_copy(acc.at[sl, :], out1.at[sl, :])


_edge_call = pl.kernel(
    _edge_body,
    out_type=(
        jax.ShapeDtypeStruct((NP, ACC_W), jnp.float32),
        jax.ShapeDtypeStruct((NP, ACC_W), jnp.float32),
    ),
    mesh=plsc.VectorSubcoreMesh(core_axis_name="c", subcore_axis_name="s",
                                num_cores=2, num_subcores=16),
    compiler_params=pltpu.CompilerParams(needs_layout_passes=False,
                                         use_tc_tiling_on_sc=False,
                                         skip_device_barrier=True),
    scratch_types=(
        [pltpu.VMEM((NP,), jnp.float32)] * 2
        + [pltpu.VMEM((WPB * WSZ,), jnp.int32)] * 2
        + [pltpu.VMEM((WSZ,), jnp.int32)] * 3
        + [pltpu.VMEM((WSZ, HD), jnp.float32)] * 3
        + [pltpu.VMEM((WSZ, ACC_W), jnp.float32)] * 3
        + [pltpu.VMEM((WSZ,), jnp.float32)]
        + [pltpu.VMEM_SHARED((NP, ACC_W), jnp.float32)]
        + [pltpu.SemaphoreType.DMA] * 7
    ),
)


def kernel(x, edge_index, params):
    p = params
    pade = NEP - NE
    pidx = jnp.arange(pade, dtype=jnp.int32)
    src = jnp.concatenate([edge_index[0], (pidx * 97) % NN])
    dst = jnp.concatenate([edge_index[1], NN + (pidx % (NP - NN))])
    xp = jnp.zeros((NP, D), jnp.float32).at[:NN].set(x)

    r2 = lambda a: a.reshape(1, D)
    zW = jnp.zeros((D, D), jnp.float32)
    zb = jnp.zeros((1, D), jnp.float32)

    h0, h1, ss, sd = _mm_call(xp, p['conv1_W'], r2(p['conv1_asrc']), r2(p['conv1_adst']))
    prev = xp
    for i in range(1, 5):
        o0, o1 = _edge_call(h0, h1, ss, sd, src, dst)
        bn = (r2(p['bn%d_g' % i]), r2(p['bn%d_b' % i]),
              r2(p['bn%d_m' % i]), r2(p['bn%d_v' % i]))
        pW = zW if i == 1 else p['proj%d_W' % i]
        pb = zb if i == 1 else r2(p['proj%d_b' % i])
        j = i + 1
        prev, h0, h1, ss, sd = _fuse_call(
            o0, o1, r2(p['conv%d_b' % i]), *bn, prev, pW, pb,
            p['conv%d_W' % j], r2(p['conv%d_asrc' % j]), r2(p['conv%d_adst' % j]))

    o0, o1 = _edge_call(h0, h1, ss, sd, src, dst)
    bn5 = (r2(p['bn5_g']), r2(p['bn5_b']), r2(p['bn5_m']), r2(p['bn5_v']))
    out = _fuse5_call(o0, o1, r2(p['conv5_b']), *bn5, prev,
                      p['proj5_W'], r2(p['proj5_b']),
                      p['head_W1'], p['head_b1'][None, :],
                      p['headbn_g'][None, :], p['headbn_b'][None, :],
                      p['headbn_m'][None, :], p['headbn_v'][None, :],
                      p['head_W2'], p['head_b2'][None, :])
    return out.reshape(-1)
